# Initial kernel scaffold; baseline (speedup 1.0000x reference)
import jax
import jax.numpy as jnp
from jax.experimental import pallas as pl


def kernel(edge_index, node_graph_ids, W1, W2, Wfc):
    raise NotImplementedError("write your pallas kernel here")


# speedup vs baseline: 34.9736x; 34.9736x over previous
import functools

import jax
import jax.numpy as jnp
from jax import lax
from jax.experimental import pallas as pl
from jax.experimental.pallas import tpu as pltpu
from jax.experimental.pallas import tpu_sc as plsc

N = 10000
E = 160000
H = 256
C = 8
G = 64

NS = 16
NP = 10240
CH = NP // NS
EPT = E // NS
NV_E = EPT // 16
NV_C = CH // 16
GP = 128
NV_G = GP // 16

_f32 = jnp.float32


def _rsqrt16(x):
    i = plsc.bitcast(x, jnp.int32)
    i = jnp.full((16,), 0x5F3759DF, jnp.int32) - lax.shift_right_logical(
        i, jnp.full((16,), 1, jnp.int32))
    y = plsc.bitcast(i, _f32)
    for _ in range(3):
        y = y * (1.5 - 0.5 * x * y * y)
    return y


def _sc_graph(src, dst, gid_pad):
    mesh = plsc.VectorSubcoreMesh(
        core_axis_name="c", subcore_axis_name="s", num_cores=2, num_subcores=NS)

    @functools.partial(
        pl.kernel,
        out_type=(jax.ShapeDtypeStruct((G,), _f32),
                  jax.ShapeDtypeStruct((G,), _f32)),
        mesh=mesh,
        compiler_params=pltpu.CompilerParams(needs_layout_passes=False),
        scratch_types=[
            pltpu.VMEM((EPT,), jnp.int32),
            pltpu.VMEM((EPT,), jnp.int32),
            pltpu.VMEM((CH,), jnp.int32),
            pltpu.VMEM((NP,), _f32),
            pltpu.VMEM((NP,), _f32),
            pltpu.VMEM((NP,), _f32),
            pltpu.VMEM((NS, CH), _f32),
            pltpu.VMEM((CH,), _f32),
            pltpu.VMEM((CH,), _f32),
            pltpu.VMEM((CH,), _f32),
            pltpu.VMEM((GP,), _f32),
            pltpu.VMEM((GP,), _f32),
            pltpu.VMEM((NS, GP), _f32),
            pltpu.VMEM_SHARED((NS, NP), _f32),
            pltpu.VMEM_SHARED((NS, NP), _f32),
            pltpu.VMEM_SHARED((NP,), _f32),
            pltpu.VMEM_SHARED((NS, GP), _f32),
            pltpu.VMEM_SHARED((NS, GP), _f32),
        ],
    )
    def run(src_h, dst_h, gid_h, u_out, c_out,
            src_v, dst_v, gid_v, acc_a, acc_b, node_v, slab, ns_c, nd_c, u_c,
            accG, cntG, slabG, mat_a, mat_b, vec_sh, matG, matC):
        s = lax.axis_index("s")
        c = lax.axis_index("c")
        ones16 = jnp.ones((16,), _f32)
        zeros16 = jnp.zeros((16,), _f32)

        def off16(i):
            return pl.ds(pl.multiple_of(i * 16, 16), 16)

        def zero_ref(ref, nv):
            def bd(i, _):
                ref[off16(i)] = zeros16
                return 0
            lax.fori_loop(0, nv, bd, 0)

        def reduce_rows(mat, nv, out_fn):
            def bd(i, _):
                acc = slab[0, off16(i)]
                for r in range(1, NS):
                    acc = acc + slab[r, off16(i)]
                out_fn(i, acc)
                return 0
            pltpu.sync_copy(mat, slab)
            lax.fori_loop(0, nv, bd, 0)

        my_nodes = pl.ds(pl.multiple_of(s * CH, 8), CH)

        pltpu.sync_copy(src_h.at[pl.ds(pl.multiple_of(s * EPT, 8), EPT)], src_v)
        pltpu.sync_copy(dst_h.at[pl.ds(pl.multiple_of(s * EPT, 8), EPT)], dst_v)
        pltpu.sync_copy(gid_h.at[my_nodes], gid_v)

        zero_ref(acc_a, NP // 16)
        zero_ref(acc_b, NP // 16)

        def deg_body(i, _):
            si = src_v[off16(i)]
            di = dst_v[off16(i)]
            plsc.addupdate_scatter(acc_a, [si], ones16)
            plsc.addupdate_scatter(acc_b, [di], ones16)
            return 0
        lax.fori_loop(0, NV_E, deg_body, 0)

        pltpu.sync_copy(acc_a, mat_a.at[s])
        pltpu.sync_copy(acc_b, mat_b.at[s])
        plsc.subcore_barrier()

        def ns_fn(i, v):
            ns_c[off16(i)] = _rsqrt16(jnp.maximum(v, 1.0))
        reduce_rows(mat_a.at[:, my_nodes], NV_C, ns_fn)

        def nd_s0_fn(i, v):
            nd = _rsqrt16(jnp.maximum(v, 1.0))
            nd_c[off16(i)] = nd
            u_c[off16(i)] = v * ns_c[off16(i)]
        reduce_rows(mat_b.at[:, my_nodes], NV_C, nd_s0_fn)

        pltpu.sync_copy(u_c, vec_sh.at[my_nodes])
        plsc.subcore_barrier()

        pltpu.sync_copy(vec_sh, node_v)
        zero_ref(acc_a, NP // 16)

        def edge_body(i, _):
            si = src_v[off16(i)]
            di = dst_v[off16(i)]
            v = plsc.load_gather(node_v, [si])
            plsc.addupdate_scatter(acc_a, [di], v)
            return 0
        lax.fori_loop(0, NV_E, edge_body, 0)

        pltpu.sync_copy(acc_a, mat_a.at[s])
        plsc.subcore_barrier()

        def p_fn(i, v):
            u_c[off16(i)] = v * nd_c[off16(i)] * ns_c[off16(i)]
        reduce_rows(mat_a.at[:, my_nodes], NV_C, p_fn)

        pltpu.sync_copy(u_c, vec_sh.at[my_nodes])
        plsc.subcore_barrier()

        pltpu.sync_copy(vec_sh, node_v)
        zero_ref(acc_a, NP // 16)
        lax.fori_loop(0, NV_E, edge_body, 0)

        pltpu.sync_copy(acc_a, mat_a.at[s])
        plsc.subcore_barrier()

        def u_fn(i, v):
            u_c[off16(i)] = v * nd_c[off16(i)]
        reduce_rows(mat_a.at[:, my_nodes], NV_C, u_fn)

        zero_ref(accG, NV_G)
        zero_ref(cntG, NV_G)

        def ro_body(i, _):
            g = gid_v[off16(i)]
            plsc.addupdate_scatter(accG, [g], u_c[off16(i)])
            plsc.addupdate_scatter(cntG, [g], ones16)
            return 0
        lax.fori_loop(0, NV_C, ro_body, 0)

        pltpu.sync_copy(accG, matG.at[s])
        pltpu.sync_copy(cntG, matC.at[s])
        plsc.subcore_barrier()

        @pl.when(jnp.logical_and(s == 0, c == 0))
        def _():
            def redG(mat, out_ref):
                def bd(i, _):
                    acc = slabG[0, off16(i)]
                    for r in range(1, NS):
                        acc = acc + slabG[r, off16(i)]
                    out_ref[off16(i)] = acc
                    return 0
                pltpu.sync_copy(mat, slabG)
                lax.fori_loop(0, NV_G, bd, 0)
            redG(matG, accG)
            redG(matC, cntG)
            pltpu.sync_copy(accG.at[pl.ds(0, G)], u_out)
            pltpu.sync_copy(cntG.at[pl.ds(0, G)], c_out)

    return run(src, dst, gid_pad)


def _tc_head(u_sum, cnt, W1, W2, Wfc):
    def body(u_ref, c_ref, w1_ref, w2_ref, wfc_ref, o_ref):
        a = u_ref[...] / jnp.maximum(c_ref[...], 1.0)
        r1 = jnp.maximum(w1_ref[...], 0.0)
        q = jnp.maximum(
            jnp.dot(r1, w2_ref[...], preferred_element_type=_f32), 0.0)
        b = jnp.dot(q, wfc_ref[...], preferred_element_type=_f32)
        o_ref[...] = a * b

    return pl.pallas_call(
        body, out_shape=jax.ShapeDtypeStruct((G, C), _f32),
    )(u_sum.reshape(G, 1), cnt.reshape(G, 1), W1, W2, Wfc)


def kernel(edge_index, node_graph_ids, W1, W2, Wfc):
    src = edge_index[0]
    dst = edge_index[1]
    gid_pad = jnp.concatenate(
        [node_graph_ids.astype(jnp.int32),
         jnp.full((NP - N,), G, jnp.int32)])
    u_sum, cnt = _sc_graph(src, dst, gid_pad)
    return _tc_head(u_sum, cnt, W1, W2, Wfc)


# speedup vs baseline: 56.1455x; 1.6054x over previous
import functools

import jax
import jax.numpy as jnp
from jax import lax
from jax.experimental import pallas as pl
from jax.experimental.pallas import tpu as pltpu
from jax.experimental.pallas import tpu_sc as plsc

N = 10000
E = 160000
H = 256
C = 8
G = 64

NS = 16
NP = 10240
CH = NP // NS
EPT = E // NS
NV_E = EPT // 16
NV_C = CH // 16
GP = 128
NV_G = GP // 16

_f32 = jnp.float32


def _rsqrt16(x):
    i = plsc.bitcast(x, jnp.int32)
    i = jnp.full((16,), 0x5F3759DF, jnp.int32) - lax.shift_right_logical(
        i, jnp.full((16,), 1, jnp.int32))
    y = plsc.bitcast(i, _f32)
    for _ in range(3):
        y = y * (1.5 - 0.5 * x * y * y)
    return y


def _sc_graph(edge_index, node_graph_ids):
    mesh = plsc.VectorSubcoreMesh(
        core_axis_name="c", subcore_axis_name="s", num_cores=2, num_subcores=NS)

    @functools.partial(
        pl.kernel,
        out_type=(jax.ShapeDtypeStruct((G,), _f32),
                  jax.ShapeDtypeStruct((G,), _f32)),
        mesh=mesh,
        compiler_params=pltpu.CompilerParams(needs_layout_passes=False),
        scratch_types=[
            pltpu.VMEM((EPT,), jnp.int32),
            pltpu.VMEM((EPT,), jnp.int32),
            pltpu.VMEM((CH,), jnp.int32),
            pltpu.VMEM((NP,), _f32),
            pltpu.VMEM((NP,), _f32),
            pltpu.VMEM((NP,), _f32),
            pltpu.VMEM((NS, CH), _f32),
            pltpu.VMEM((CH,), _f32),
            pltpu.VMEM((CH,), _f32),
            pltpu.VMEM((CH,), _f32),
            pltpu.VMEM((GP,), _f32),
            pltpu.VMEM((GP,), _f32),
            pltpu.VMEM((NS, GP), _f32),
            pltpu.VMEM_SHARED((NS, NP), _f32),
            pltpu.VMEM_SHARED((NS, NP), _f32),
            pltpu.VMEM_SHARED((NP,), _f32),
            pltpu.VMEM_SHARED((NS, GP), _f32),
            pltpu.VMEM_SHARED((NS, GP), _f32),
        ],
    )
    def run(ei_h, gid_h, u_out, c_out,
            src_v, dst_v, gid_v, acc_a, acc_b, node_v, slab, ns_c, nd_c, u_c,
            accG, cntG, slabG, mat_a, mat_b, vec_sh, matG, matC):
        s = lax.axis_index("s")
        c = lax.axis_index("c")
        ones16 = jnp.ones((16,), _f32)
        zeros16 = jnp.zeros((16,), _f32)

        def off16(i):
            return pl.ds(pl.multiple_of(i * 16, 16), 16)

        def zero_ref(ref, nv):
            @plsc.parallel_loop(0, nv, 1, unroll=8)
            def _(i):
                ref[off16(i)] = zeros16

        def reduce_rows(mat, nv, out_fn):
            pltpu.sync_copy(mat, slab)

            @plsc.parallel_loop(0, nv, 1, unroll=2)
            def _(i):
                acc = slab[0, off16(i)]
                for r in range(1, NS):
                    acc = acc + slab[r, off16(i)]
                out_fn(i, acc)

        my_nodes = pl.ds(pl.multiple_of(s * CH, 8), CH)

        pltpu.sync_copy(ei_h.at[pl.ds(pl.multiple_of(s * EPT, 8), EPT)], src_v)
        pltpu.sync_copy(ei_h.at[pl.ds(pl.multiple_of(E + s * EPT, 8), EPT)], dst_v)

        @pl.when(s < NS - 1)
        def _():
            pltpu.sync_copy(gid_h.at[my_nodes], gid_v)

        @pl.when(s == NS - 1)
        def _():
            tail = N - (NS - 1) * CH
            pltpu.sync_copy(gid_h.at[pl.ds((NS - 1) * CH, tail)],
                            gid_v.at[pl.ds(0, tail)])
            for i in range(tail // 16, NV_C):
                gid_v[off16(i)] = jnp.full((16,), G, jnp.int32)

        zero_ref(acc_a, NP // 16)
        zero_ref(acc_b, NP // 16)

        @plsc.parallel_loop(0, NV_E, 1, unroll=8)
        def _(i):
            plsc.addupdate_scatter(acc_a, [src_v[off16(i)]], ones16)
            plsc.addupdate_scatter(acc_b, [dst_v[off16(i)]], ones16)

        pltpu.sync_copy(acc_a, mat_a.at[s])
        pltpu.sync_copy(acc_b, mat_b.at[s])
        plsc.subcore_barrier()

        def ns_fn(i, v):
            ns_c[off16(i)] = _rsqrt16(jnp.maximum(v, 1.0))
        reduce_rows(mat_a.at[:, my_nodes], NV_C, ns_fn)

        def nd_s0_fn(i, v):
            nd = _rsqrt16(jnp.maximum(v, 1.0))
            nd_c[off16(i)] = nd
            u_c[off16(i)] = v * ns_c[off16(i)]
        reduce_rows(mat_b.at[:, my_nodes], NV_C, nd_s0_fn)

        pltpu.sync_copy(u_c, vec_sh.at[my_nodes])
        plsc.subcore_barrier()

        pltpu.sync_copy(vec_sh, node_v)
        zero_ref(acc_a, NP // 16)

        def edge_pass():
            @plsc.parallel_loop(0, NV_E, 1, unroll=8)
            def _(i):
                v = plsc.load_gather(node_v, [src_v[off16(i)]])
                plsc.addupdate_scatter(acc_a, [dst_v[off16(i)]], v)

        edge_pass()
        pltpu.sync_copy(acc_a, mat_a.at[s])
        plsc.subcore_barrier()

        def p_fn(i, v):
            u_c[off16(i)] = v * nd_c[off16(i)] * ns_c[off16(i)]
        reduce_rows(mat_a.at[:, my_nodes], NV_C, p_fn)

        pltpu.sync_copy(u_c, vec_sh.at[my_nodes])
        plsc.subcore_barrier()

        pltpu.sync_copy(vec_sh, node_v)
        zero_ref(acc_a, NP // 16)
        edge_pass()

        pltpu.sync_copy(acc_a, mat_a.at[s])
        plsc.subcore_barrier()

        def u_fn(i, v):
            u_c[off16(i)] = v * nd_c[off16(i)]
        reduce_rows(mat_a.at[:, my_nodes], NV_C, u_fn)

        zero_ref(accG, NV_G)
        zero_ref(cntG, NV_G)

        @plsc.parallel_loop(0, NV_C, 1, unroll=4)
        def _(i):
            g = gid_v[off16(i)]
            plsc.addupdate_scatter(accG, [g], u_c[off16(i)])
            plsc.addupdate_scatter(cntG, [g], ones16)

        pltpu.sync_copy(accG, matG.at[s])
        pltpu.sync_copy(cntG, matC.at[s])
        plsc.subcore_barrier()

        @pl.when(jnp.logical_and(s == 0, c == 0))
        def _():
            def redG(mat, out_ref):
                pltpu.sync_copy(mat, slabG)

                @plsc.parallel_loop(0, NV_G, 1, unroll=2)
                def _(i):
                    acc = slabG[0, off16(i)]
                    for r in range(1, NS):
                        acc = acc + slabG[r, off16(i)]
                    out_ref[off16(i)] = acc
            redG(matG, accG)
            redG(matC, cntG)
            pltpu.sync_copy(accG.at[pl.ds(0, G)], u_out)
            pltpu.sync_copy(cntG.at[pl.ds(0, G)], c_out)

    return run(edge_index, node_graph_ids)


def _tc_head(u_sum, cnt, W1, W2, Wfc):
    def body(u_ref, c_ref, w1_ref, w2_ref, wfc_ref, o_ref):
        a = u_ref[...] / jnp.maximum(c_ref[...], 1.0)
        r1 = jnp.maximum(w1_ref[...], 0.0)
        q = jnp.maximum(
            jnp.dot(r1, w2_ref[...], preferred_element_type=_f32), 0.0)
        b = jnp.dot(q, wfc_ref[...], preferred_element_type=_f32)
        o_ref[...] = a * b

    return pl.pallas_call(
        body, out_shape=jax.ShapeDtypeStruct((G, C), _f32),
    )(u_sum.reshape(G, 1), cnt.reshape(G, 1), W1, W2, Wfc)


def kernel(edge_index, node_graph_ids, W1, W2, Wfc):
    u_sum, cnt = _sc_graph(edge_index.reshape(2 * E), node_graph_ids.astype(jnp.int32))
    return _tc_head(u_sum, cnt, W1, W2, Wfc)


# speedup vs baseline: 60.1076x; 1.0706x over previous
import functools

import jax
import jax.numpy as jnp
from jax import lax
from jax.experimental import pallas as pl
from jax.experimental.pallas import tpu as pltpu
from jax.experimental.pallas import tpu_sc as plsc

N = 10000
E = 160000
H = 256
C = 8
G = 64

NS = 16
NP = 10240
CH = NP // NS
EPT = 10240
EPT_L = E - 15 * EPT
NV_E = EPT // 16
NV_E_L = EPT_L // 16
NV_C = CH // 16
GP = 128
NV_G = GP // 16

_f32 = jnp.float32


def _rsqrt16(x):
    i = plsc.bitcast(x, jnp.int32)
    i = jnp.full((16,), 0x5F3759DF, jnp.int32) - lax.shift_right_logical(
        i, jnp.full((16,), 1, jnp.int32))
    y = plsc.bitcast(i, _f32)
    for _ in range(3):
        y = y * (1.5 - 0.5 * x * y * y)
    return y


def _sc_graph(edge_index, node_graph_ids):
    mesh = plsc.VectorSubcoreMesh(
        core_axis_name="c", subcore_axis_name="s", num_cores=2, num_subcores=NS)

    @functools.partial(
        pl.kernel,
        out_type=(jax.ShapeDtypeStruct((G,), _f32),
                  jax.ShapeDtypeStruct((G,), _f32)),
        mesh=mesh,
        compiler_params=pltpu.CompilerParams(needs_layout_passes=False),
        scratch_types=[
            pltpu.VMEM((2, EPT), jnp.int32),
            pltpu.VMEM((CH,), jnp.int32),
            pltpu.VMEM((NP,), _f32),
            pltpu.VMEM((NP,), _f32),
            pltpu.VMEM((NP,), _f32),
            pltpu.VMEM((NS, CH), _f32),
            pltpu.VMEM((CH,), _f32),
            pltpu.VMEM((CH,), _f32),
            pltpu.VMEM((CH,), _f32),
            pltpu.VMEM((GP,), _f32),
            pltpu.VMEM((GP,), _f32),
            pltpu.VMEM((NS, GP), _f32),
            pltpu.VMEM_SHARED((NS, NP), _f32),
            pltpu.VMEM_SHARED((NS, NP), _f32),
            pltpu.VMEM_SHARED((NP,), _f32),
            pltpu.VMEM_SHARED((NS, GP), _f32),
            pltpu.VMEM_SHARED((NS, GP), _f32),
        ],
    )
    def run(ei_h, gid_h, u_out, c_out,
            ei_v, gid_v, acc_a, acc_b, node_v, slab, ns_c, nd_c, u_c,
            accG, cntG, slabG, mat_a, mat_b, vec_sh, matG, matC):
        s = lax.axis_index("s")
        c = lax.axis_index("c")
        ones16 = jnp.ones((16,), _f32)
        zeros16 = jnp.zeros((16,), _f32)

        def off16(i):
            return pl.ds(pl.multiple_of(i * 16, 16), 16)

        def zero_ref(ref, nv):
            @plsc.parallel_loop(0, nv, 1, unroll=8)
            def _(i):
                ref[off16(i)] = zeros16

        def reduce_rows(mat, nv, out_fn):
            pltpu.sync_copy(mat, slab)

            @plsc.parallel_loop(0, nv, 1, unroll=2)
            def _(i):
                acc = slab[0, off16(i)]
                for r in range(1, NS):
                    acc = acc + slab[r, off16(i)]
                out_fn(i, acc)

        my_nodes = pl.ds(pl.multiple_of(s * CH, 8), CH)

        @pl.when(s < NS - 1)
        def _():
            pltpu.sync_copy(
                ei_h.at[:, pl.ds(pl.multiple_of(s * EPT, 512), EPT)], ei_v)

        @pl.when(s == NS - 1)
        def _():
            pltpu.sync_copy(ei_h.at[:, pl.ds((NS - 1) * EPT, EPT_L)],
                            ei_v.at[:, pl.ds(0, EPT_L)])

        @pl.when(s < NS - 1)
        def _():
            pltpu.sync_copy(gid_h.at[my_nodes], gid_v)

        @pl.when(s == NS - 1)
        def _():
            tail = N - (NS - 1) * CH
            pltpu.sync_copy(gid_h.at[pl.ds((NS - 1) * CH, tail)],
                            gid_v.at[pl.ds(0, tail)])
            for i in range(tail // 16, NV_C):
                gid_v[off16(i)] = jnp.full((16,), G, jnp.int32)

        zero_ref(acc_a, NP // 16)
        zero_ref(acc_b, NP // 16)

        def deg_body(i):
            plsc.addupdate_scatter(acc_a, [ei_v[0, off16(i)]], ones16)
            plsc.addupdate_scatter(acc_b, [ei_v[1, off16(i)]], ones16)

        plsc.parallel_loop(0, NV_E_L, 1, unroll=8)(deg_body)

        @pl.when(s < NS - 1)
        def _():
            plsc.parallel_loop(NV_E_L, NV_E, 1, unroll=8)(deg_body)

        pltpu.sync_copy(acc_a, mat_a.at[s])
        pltpu.sync_copy(acc_b, mat_b.at[s])
        plsc.subcore_barrier()

        def ns_fn(i, v):
            ns_c[off16(i)] = _rsqrt16(jnp.maximum(v, 1.0))
        reduce_rows(mat_a.at[:, my_nodes], NV_C, ns_fn)

        def nd_s0_fn(i, v):
            nd = _rsqrt16(jnp.maximum(v, 1.0))
            nd_c[off16(i)] = nd
            u_c[off16(i)] = v * ns_c[off16(i)]
        reduce_rows(mat_b.at[:, my_nodes], NV_C, nd_s0_fn)

        pltpu.sync_copy(u_c, vec_sh.at[my_nodes])
        plsc.subcore_barrier()

        pltpu.sync_copy(vec_sh, node_v)
        zero_ref(acc_a, NP // 16)

        def edge_pass():
            def body(i):
                v = plsc.load_gather(node_v, [ei_v[0, off16(i)]])
                plsc.addupdate_scatter(acc_a, [ei_v[1, off16(i)]], v)

            plsc.parallel_loop(0, NV_E_L, 1, unroll=8)(body)

            @pl.when(s < NS - 1)
            def _():
                plsc.parallel_loop(NV_E_L, NV_E, 1, unroll=8)(body)

        edge_pass()
        pltpu.sync_copy(acc_a, mat_a.at[s])
        plsc.subcore_barrier()

        def p_fn(i, v):
            u_c[off16(i)] = v * nd_c[off16(i)] * ns_c[off16(i)]
        reduce_rows(mat_a.at[:, my_nodes], NV_C, p_fn)

        pltpu.sync_copy(u_c, vec_sh.at[my_nodes])
        plsc.subcore_barrier()

        pltpu.sync_copy(vec_sh, node_v)
        zero_ref(acc_a, NP // 16)
        edge_pass()

        pltpu.sync_copy(acc_a, mat_a.at[s])
        plsc.subcore_barrier()

        def u_fn(i, v):
            u_c[off16(i)] = v * nd_c[off16(i)]
        reduce_rows(mat_a.at[:, my_nodes], NV_C, u_fn)

        zero_ref(accG, NV_G)
        zero_ref(cntG, NV_G)

        @plsc.parallel_loop(0, NV_C, 1, unroll=4)
        def _(i):
            g = gid_v[off16(i)]
            plsc.addupdate_scatter(accG, [g], u_c[off16(i)])
            plsc.addupdate_scatter(cntG, [g], ones16)

        pltpu.sync_copy(accG, matG.at[s])
        pltpu.sync_copy(cntG, matC.at[s])
        plsc.subcore_barrier()

        @pl.when(jnp.logical_and(s == 0, c == 0))
        def _():
            def redG(mat, out_ref):
                pltpu.sync_copy(mat, slabG)

                @plsc.parallel_loop(0, NV_G, 1, unroll=2)
                def _(i):
                    acc = slabG[0, off16(i)]
                    for r in range(1, NS):
                        acc = acc + slabG[r, off16(i)]
                    out_ref[off16(i)] = acc
            redG(matG, accG)
            redG(matC, cntG)
            pltpu.sync_copy(accG.at[pl.ds(0, G)], u_out)
            pltpu.sync_copy(cntG.at[pl.ds(0, G)], c_out)

    return run(edge_index, node_graph_ids)


def _tc_head(u_sum, cnt, W1, W2, Wfc):
    def body(u_ref, c_ref, w1_ref, w2_ref, wfc_ref, o_ref):
        a = u_ref[...] / jnp.maximum(c_ref[...], 1.0)
        r1 = jnp.maximum(w1_ref[...], 0.0)
        q = jnp.maximum(
            jnp.dot(r1, w2_ref[...], preferred_element_type=_f32), 0.0)
        b = jnp.dot(q, wfc_ref[...], preferred_element_type=_f32)
        o_ref[...] = a[:, None] * b

    return pl.pallas_call(
        body, out_shape=jax.ShapeDtypeStruct((G, C), _f32),
    )(u_sum, cnt, W1, W2, Wfc)


def kernel(edge_index, node_graph_ids, W1, W2, Wfc):
    u_sum, cnt = _sc_graph(edge_index, node_graph_ids.astype(jnp.int32))
    return _tc_head(u_sum, cnt, W1, W2, Wfc)


# speedup vs baseline: 60.6467x; 1.0090x over previous
import functools

import jax
import jax.numpy as jnp
from jax import lax
from jax.experimental import pallas as pl
from jax.experimental.pallas import tpu as pltpu
from jax.experimental.pallas import tpu_sc as plsc

N = 10000
E = 160000
H = 256
C = 8
G = 64

NS = 16
NP = 10240
CH = NP // NS
EPT = 10240
EPT_L = E - 15 * EPT
NV_E = EPT // 16
NV_E_L = EPT_L // 16
NV_C = CH // 16
GP = 128
NV_G = GP // 16

_f32 = jnp.float32


def _rsqrt16(x):
    i = plsc.bitcast(x, jnp.int32)
    i = jnp.full((16,), 0x5F3759DF, jnp.int32) - lax.shift_right_logical(
        i, jnp.full((16,), 1, jnp.int32))
    y = plsc.bitcast(i, _f32)
    for _ in range(3):
        y = y * (1.5 - 0.5 * x * y * y)
    return y


def _sc_graph():
    mesh = plsc.VectorSubcoreMesh(
        core_axis_name="c", subcore_axis_name="s", num_cores=2, num_subcores=NS)

    @functools.partial(
        pl.kernel,
        out_type=jax.ShapeDtypeStruct((G * C,), _f32),
        mesh=mesh,
        compiler_params=pltpu.CompilerParams(needs_layout_passes=False),
        scratch_types=[
            pltpu.VMEM((2, EPT), jnp.int32),
            pltpu.VMEM((CH,), jnp.int32),
            pltpu.VMEM((NP,), _f32),
            pltpu.VMEM((NP,), _f32),
            pltpu.VMEM((NP,), _f32),
            pltpu.VMEM((NS, CH), _f32),
            pltpu.VMEM((CH,), _f32),
            pltpu.VMEM((CH,), _f32),
            pltpu.VMEM((CH,), _f32),
            pltpu.VMEM((GP,), _f32),
            pltpu.VMEM((GP,), _f32),
            pltpu.VMEM((16,), _f32),
            pltpu.VMEM((16,), _f32),
            pltpu.VMEM((G * C,), _f32),
            pltpu.VMEM((NS, GP), _f32),
            pltpu.VMEM_SHARED((NS, NP), _f32),
            pltpu.VMEM_SHARED((NS, NP), _f32),
            pltpu.VMEM_SHARED((NP,), _f32),
            pltpu.VMEM_SHARED((NS, GP), _f32),
            pltpu.VMEM_SHARED((NS, GP), _f32),
        ],
    )
    def run(ei_h, gid_h, b_h, out_h,
            ei_v, gid_v, acc_a, acc_b, node_v, slab, ns_c, nd_c, u_c,
            accG, cntG, b_v, b2_v, out_v, slabG,
            mat_a, mat_b, vec_sh, matG, matC):
        s = lax.axis_index("s")
        c = lax.axis_index("c")
        ones16 = jnp.ones((16,), _f32)
        zeros16 = jnp.zeros((16,), _f32)

        def off16(i):
            return pl.ds(pl.multiple_of(i * 16, 16), 16)

        def zero_ref(ref, nv):
            @plsc.parallel_loop(0, nv, 1, unroll=8)
            def _(i):
                ref[off16(i)] = zeros16

        def reduce_rows(mat, nv, out_fn):
            pltpu.sync_copy(mat, slab)

            @plsc.parallel_loop(0, nv, 1, unroll=2)
            def _(i):
                acc = slab[0, off16(i)]
                for r in range(1, NS):
                    acc = acc + slab[r, off16(i)]
                out_fn(i, acc)

        my_nodes = pl.ds(pl.multiple_of(s * CH, 8), CH)

        pltpu.sync_copy(b_h, b_v)

        @pl.when(s < NS - 1)
        def _():
            pltpu.sync_copy(
                ei_h.at[:, pl.ds(pl.multiple_of(s * EPT, 512), EPT)], ei_v)

        @pl.when(s == NS - 1)
        def _():
            pltpu.sync_copy(ei_h.at[:, pl.ds((NS - 1) * EPT, EPT_L)],
                            ei_v.at[:, pl.ds(0, EPT_L)])

        @pl.when(s < NS - 1)
        def _():
            pltpu.sync_copy(gid_h.at[my_nodes], gid_v)

        @pl.when(s == NS - 1)
        def _():
            tail = N - (NS - 1) * CH
            pltpu.sync_copy(gid_h.at[pl.ds((NS - 1) * CH, tail)],
                            gid_v.at[pl.ds(0, tail)])
            for i in range(tail // 16, NV_C):
                gid_v[off16(i)] = jnp.full((16,), G, jnp.int32)

        zero_ref(acc_a, NP // 16)
        zero_ref(acc_b, NP // 16)

        def deg_body(i):
            plsc.addupdate_scatter(acc_a, [ei_v[0, off16(i)]], ones16)
            plsc.addupdate_scatter(acc_b, [ei_v[1, off16(i)]], ones16)

        plsc.parallel_loop(0, NV_E_L, 1, unroll=8)(deg_body)

        @pl.when(s < NS - 1)
        def _():
            plsc.parallel_loop(NV_E_L, NV_E, 1, unroll=8)(deg_body)

        pltpu.sync_copy(acc_a, mat_a.at[s])
        pltpu.sync_copy(acc_b, mat_b.at[s])
        plsc.subcore_barrier()

        def ns_fn(i, v):
            ns_c[off16(i)] = _rsqrt16(jnp.maximum(v, 1.0))
        reduce_rows(mat_a.at[:, my_nodes], NV_C, ns_fn)

        def nd_s0_fn(i, v):
            nd = _rsqrt16(jnp.maximum(v, 1.0))
            nd_c[off16(i)] = nd
            u_c[off16(i)] = v * ns_c[off16(i)]
        reduce_rows(mat_b.at[:, my_nodes], NV_C, nd_s0_fn)

        pltpu.sync_copy(u_c, vec_sh.at[my_nodes])
        plsc.subcore_barrier()

        pltpu.sync_copy(vec_sh, node_v)
        zero_ref(acc_a, NP // 16)

        def edge_pass():
            def body(i):
                v = plsc.load_gather(node_v, [ei_v[0, off16(i)]])
                plsc.addupdate_scatter(acc_a, [ei_v[1, off16(i)]], v)

            plsc.parallel_loop(0, NV_E_L, 1, unroll=8)(body)

            @pl.when(s < NS - 1)
            def _():
                plsc.parallel_loop(NV_E_L, NV_E, 1, unroll=8)(body)

        edge_pass()
        pltpu.sync_copy(acc_a, mat_a.at[s])
        plsc.subcore_barrier()

        def p_fn(i, v):
            u_c[off16(i)] = v * nd_c[off16(i)] * ns_c[off16(i)]
        reduce_rows(mat_a.at[:, my_nodes], NV_C, p_fn)

        pltpu.sync_copy(u_c, vec_sh.at[my_nodes])
        plsc.subcore_barrier()

        pltpu.sync_copy(vec_sh, node_v)
        zero_ref(acc_a, NP // 16)
        edge_pass()

        pltpu.sync_copy(acc_a, mat_a.at[s])
        plsc.subcore_barrier()

        def u_fn(i, v):
            u_c[off16(i)] = v * nd_c[off16(i)]
        reduce_rows(mat_a.at[:, my_nodes], NV_C, u_fn)

        zero_ref(accG, NV_G)
        zero_ref(cntG, NV_G)

        @plsc.parallel_loop(0, NV_C, 1, unroll=4)
        def _(i):
            g = gid_v[off16(i)]
            plsc.addupdate_scatter(accG, [g], u_c[off16(i)])
            plsc.addupdate_scatter(cntG, [g], ones16)

        pltpu.sync_copy(accG, matG.at[s])
        pltpu.sync_copy(cntG, matC.at[s])
        plsc.subcore_barrier()

        @pl.when(jnp.logical_and(s == 0, c == 0))
        def _():
            def redG(mat, out_ref):
                pltpu.sync_copy(mat, slabG)

                @plsc.parallel_loop(0, NV_G, 1, unroll=2)
                def _(i):
                    acc = slabG[0, off16(i)]
                    for r in range(1, NS):
                        acc = acc + slabG[r, off16(i)]
                    out_ref[off16(i)] = acc
            redG(matG, accG)
            redG(matC, cntG)
            iota16 = lax.iota(jnp.int32, 16)
            lo8 = iota16 & 7
            bv = b_v[off16(0)]
            b2_v[off16(0)] = zeros16
            plsc.addupdate_scatter(b2_v, [lo8], bv)
            plsc.addupdate_scatter(b2_v, [lo8 + 8], bv)
            b2 = b2_v[off16(0)]
            lohalf = iota16 < 8
            for vb in range(G // 16):
                av = accG[off16(vb)] / jnp.maximum(cntG[off16(vb)], 1.0)
                for h in range(8):
                    a0 = jnp.broadcast_to(av[2 * h], (16,))
                    a1 = jnp.broadcast_to(av[2 * h + 1], (16,))
                    out_v[off16(vb * 8 + h)] = jnp.where(lohalf, a0, a1) * b2
            pltpu.sync_copy(out_v, out_h)

    return run


def _tc_bvec(W1, W2, Wfc):
    def body(w1_ref, w2_ref, wfc_ref, o_ref):
        r1 = jnp.maximum(w1_ref[...], 0.0)
        q = jnp.maximum(
            jnp.dot(r1, w2_ref[...], preferred_element_type=_f32), 0.0)
        b = jnp.dot(q, wfc_ref[...], preferred_element_type=_f32)
        o_ref[...] = jnp.concatenate([b, jnp.zeros((1, 16 - C), _f32)], axis=1)

    return pl.pallas_call(
        body, out_shape=jax.ShapeDtypeStruct((1, 16), _f32),
    )(W1, W2, Wfc).reshape(16)


def kernel(edge_index, node_graph_ids, W1, W2, Wfc):
    b = _tc_bvec(W1, W2, Wfc)
    out_flat = _sc_graph()(edge_index, node_graph_ids.astype(jnp.int32), b)
    return out_flat.reshape(G, C)


# speedup vs baseline: 63.2556x; 1.0430x over previous
import functools

import jax
import jax.numpy as jnp
from jax import lax
from jax.experimental import pallas as pl
from jax.experimental.pallas import tpu as pltpu
from jax.experimental.pallas import tpu_sc as plsc

N = 10000
E = 160000
H = 256
C = 8
G = 64

NS = 16
NP = 10240
CH = NP // NS
EPT = 10240
EPT_L = E - 15 * EPT
NV_E = EPT // 16
NV_E_L = EPT_L // 16
NV_C = CH // 16
GP = 128
NV_G = GP // 16

_f32 = jnp.float32


def _rsqrt16(x):
    i = plsc.bitcast(x, jnp.int32)
    i = jnp.full((16,), 0x5F3759DF, jnp.int32) - lax.shift_right_logical(
        i, jnp.full((16,), 1, jnp.int32))
    y = plsc.bitcast(i, _f32)
    for _ in range(3):
        y = y * (1.5 - 0.5 * x * y * y)
    return y


def _sc_graph():
    mesh = plsc.VectorSubcoreMesh(
        core_axis_name="c", subcore_axis_name="s", num_cores=2, num_subcores=NS)

    @functools.partial(
        pl.kernel,
        out_type=(jax.ShapeDtypeStruct((G,), _f32),
                  jax.ShapeDtypeStruct((G,), _f32)),
        mesh=mesh,
        compiler_params=pltpu.CompilerParams(needs_layout_passes=False),
        scratch_types=[
            pltpu.VMEM((2, EPT), jnp.int32),
            pltpu.VMEM((CH,), jnp.int32),
            pltpu.VMEM((NP,), _f32),
            pltpu.VMEM((NP,), _f32),
            pltpu.VMEM((NP,), _f32),
            pltpu.VMEM((NS, CH), _f32),
            pltpu.VMEM((CH,), _f32),
            pltpu.VMEM((CH,), _f32),
            pltpu.VMEM((CH,), _f32),
            pltpu.VMEM((GP,), _f32),
            pltpu.VMEM((GP,), _f32),
            pltpu.VMEM((NS, GP), _f32),
            pltpu.VMEM_SHARED((NS, NP), _f32),
            pltpu.VMEM_SHARED((NS, NP), _f32),
            pltpu.VMEM_SHARED((NP,), _f32),
            pltpu.VMEM_SHARED((NS, GP), _f32),
            pltpu.VMEM_SHARED((NS, GP), _f32),
        ],
    )
    def run(ei_h, gid_h, u_out, c_out,
            ei_v, gid_v, acc_a, acc_b, node_v, slab, ns_c, nd_c, u_c,
            accG, cntG, slabG, mat_a, mat_b, vec_sh, matG, matC):
        s = lax.axis_index("s")
        c = lax.axis_index("c")
        ones16 = jnp.ones((16,), _f32)
        zeros16 = jnp.zeros((16,), _f32)

        def off16(i):
            return pl.ds(pl.multiple_of(i * 16, 16), 16)

        def zero_ref(ref, nv):
            @plsc.parallel_loop(0, nv, 1, unroll=8)
            def _(i):
                ref[off16(i)] = zeros16

        def reduce_rows(mat, nv, out_fn):
            pltpu.sync_copy(mat, slab)

            @plsc.parallel_loop(0, nv, 1, unroll=2)
            def _(i):
                acc = slab[0, off16(i)]
                for r in range(1, NS):
                    acc = acc + slab[r, off16(i)]
                out_fn(i, acc)

        my_nodes = pl.ds(pl.multiple_of(s * CH, 8), CH)

        @pl.when(s < NS - 1)
        def _():
            pltpu.sync_copy(
                ei_h.at[:, pl.ds(pl.multiple_of(s * EPT, 512), EPT)], ei_v)

        @pl.when(s == NS - 1)
        def _():
            pltpu.sync_copy(ei_h.at[:, pl.ds((NS - 1) * EPT, EPT_L)],
                            ei_v.at[:, pl.ds(0, EPT_L)])

        @pl.when(s < NS - 1)
        def _():
            pltpu.sync_copy(gid_h.at[my_nodes], gid_v)

        @pl.when(s == NS - 1)
        def _():
            tail = N - (NS - 1) * CH
            pltpu.sync_copy(gid_h.at[pl.ds((NS - 1) * CH, tail)],
                            gid_v.at[pl.ds(0, tail)])
            for i in range(tail // 16, NV_C):
                gid_v[off16(i)] = jnp.full((16,), G, jnp.int32)

        zero_ref(acc_a, NP // 16)
        zero_ref(acc_b, NP // 16)

        def deg_body(i):
            plsc.addupdate_scatter(acc_a, [ei_v[0, off16(i)]], ones16)
            plsc.addupdate_scatter(acc_b, [ei_v[1, off16(i)]], ones16)

        plsc.parallel_loop(0, NV_E_L, 1, unroll=8)(deg_body)

        @pl.when(s < NS - 1)
        def _():
            plsc.parallel_loop(NV_E_L, NV_E, 1, unroll=8)(deg_body)

        pltpu.sync_copy(acc_a, mat_a.at[s])
        pltpu.sync_copy(acc_b, mat_b.at[s])
        plsc.subcore_barrier()

        def ns_fn(i, v):
            ns_c[off16(i)] = _rsqrt16(jnp.maximum(v, 1.0))
        reduce_rows(mat_a.at[:, my_nodes], NV_C, ns_fn)

        def nd_s0_fn(i, v):
            nd = _rsqrt16(jnp.maximum(v, 1.0))
            nd_c[off16(i)] = nd
            u_c[off16(i)] = v * ns_c[off16(i)]
        reduce_rows(mat_b.at[:, my_nodes], NV_C, nd_s0_fn)

        pltpu.sync_copy(u_c, vec_sh.at[my_nodes])
        plsc.subcore_barrier()

        pltpu.sync_copy(vec_sh, node_v)
        zero_ref(acc_a, NP // 16)

        def edge_pass():
            def body(i):
                v = plsc.load_gather(node_v, [ei_v[0, off16(i)]])
                plsc.addupdate_scatter(acc_a, [ei_v[1, off16(i)]], v)

            plsc.parallel_loop(0, NV_E_L, 1, unroll=8)(body)

            @pl.when(s < NS - 1)
            def _():
                plsc.parallel_loop(NV_E_L, NV_E, 1, unroll=8)(body)

        edge_pass()
        pltpu.sync_copy(acc_a, mat_a.at[s])
        plsc.subcore_barrier()

        def p_fn(i, v):
            u_c[off16(i)] = v * nd_c[off16(i)] * ns_c[off16(i)]
        reduce_rows(mat_a.at[:, my_nodes], NV_C, p_fn)

        pltpu.sync_copy(u_c, vec_sh.at[my_nodes])
        plsc.subcore_barrier()

        pltpu.sync_copy(vec_sh, node_v)
        zero_ref(acc_a, NP // 16)
        edge_pass()

        pltpu.sync_copy(acc_a, mat_a.at[s])
        plsc.subcore_barrier()

        def u_fn(i, v):
            u_c[off16(i)] = v * nd_c[off16(i)]
        reduce_rows(mat_a.at[:, my_nodes], NV_C, u_fn)

        zero_ref(accG, NV_G)
        zero_ref(cntG, NV_G)

        @plsc.parallel_loop(0, NV_C, 1, unroll=4)
        def _(i):
            g = gid_v[off16(i)]
            plsc.addupdate_scatter(accG, [g], u_c[off16(i)])
            plsc.addupdate_scatter(cntG, [g], ones16)

        pltpu.sync_copy(accG, matG.at[s])
        pltpu.sync_copy(cntG, matC.at[s])
        plsc.subcore_barrier()

        @pl.when(jnp.logical_and(s == 0, c == 0))
        def _():
            def redG(mat, out_ref):
                pltpu.sync_copy(mat, slabG)

                @plsc.parallel_loop(0, NV_G, 1, unroll=2)
                def _(i):
                    acc = slabG[0, off16(i)]
                    for r in range(1, NS):
                        acc = acc + slabG[r, off16(i)]
                    out_ref[off16(i)] = acc
            redG(matG, accG)
            redG(matC, cntG)
            pltpu.sync_copy(accG.at[pl.ds(0, G)], u_out)
            pltpu.sync_copy(cntG.at[pl.ds(0, G)], c_out)

    return run


def _tc_bvec(W1, W2, Wfc):
    def body(w1_ref, w2_ref, wfc_ref, o_ref):
        r1 = jnp.maximum(w1_ref[...], 0.0)
        q = jnp.maximum(
            jnp.dot(r1, w2_ref[...], preferred_element_type=_f32), 0.0)
        b = jnp.dot(q, wfc_ref[...], preferred_element_type=_f32)
        o_ref[...] = jnp.concatenate([b, jnp.zeros((1, 16 - C), _f32)], axis=1)

    return pl.pallas_call(
        body, out_shape=jax.ShapeDtypeStruct((1, 16), _f32),
    )(W1, W2, Wfc).reshape(16)


def kernel(edge_index, node_graph_ids, W1, W2, Wfc):
    b = _tc_bvec(W1, W2, Wfc)
    u_sum, cnt = _sc_graph()(edge_index, node_graph_ids.astype(jnp.int32))
    a = u_sum / jnp.maximum(cnt, 1.0)
    return a[:, None] * b[None, :C]


# speedup vs baseline: 64.3980x; 1.0181x over previous
import functools

import jax
import jax.numpy as jnp
from jax import lax
from jax.experimental import pallas as pl
from jax.experimental.pallas import tpu as pltpu
from jax.experimental.pallas import tpu_sc as plsc

N = 10000
E = 160000
H = 256
C = 8
G = 64

NS = 16
NP = 10240
CH = NP // NS
EPT = 10240
EPT_L = E - 15 * EPT
NV_E = EPT // 16
NV_E_L = EPT_L // 16
NV_C = CH // 16
GP = 128
NV_G = GP // 16

_f32 = jnp.float32


def _rsqrt16(x):
    i = plsc.bitcast(x, jnp.int32)
    i = jnp.full((16,), 0x5F3759DF, jnp.int32) - lax.shift_right_logical(
        i, jnp.full((16,), 1, jnp.int32))
    y = plsc.bitcast(i, _f32)
    for _ in range(3):
        y = y * (1.5 - 0.5 * x * y * y)
    return y


def _sc_graph():
    mesh = plsc.VectorSubcoreMesh(
        core_axis_name="c", subcore_axis_name="s", num_cores=2, num_subcores=NS)

    @functools.partial(
        pl.kernel,
        out_type=(jax.ShapeDtypeStruct((G,), _f32),
                  jax.ShapeDtypeStruct((G,), _f32)),
        mesh=mesh,
        compiler_params=pltpu.CompilerParams(needs_layout_passes=False),
        scratch_types=[
            pltpu.VMEM((2, EPT), jnp.int32),
            pltpu.VMEM((CH,), jnp.int32),
            pltpu.VMEM((NP,), _f32),
            pltpu.VMEM((NP,), _f32),
            pltpu.VMEM((NP,), _f32),
            pltpu.VMEM((NS, CH), _f32),
            pltpu.VMEM((CH,), _f32),
            pltpu.VMEM((CH,), _f32),
            pltpu.VMEM((CH,), _f32),
            pltpu.VMEM((GP,), _f32),
            pltpu.VMEM((GP,), _f32),
            pltpu.VMEM((NS, GP), _f32),
            pltpu.SemaphoreType.DMA,
            pltpu.SemaphoreType.DMA,
            pltpu.SemaphoreType.DMA,
            pltpu.VMEM_SHARED((NS, NP), _f32),
            pltpu.VMEM_SHARED((NS, NP), _f32),
            pltpu.VMEM_SHARED((NP,), _f32),
            pltpu.VMEM_SHARED((NS, GP), _f32),
            pltpu.VMEM_SHARED((NS, GP), _f32),
        ],
    )
    def run(ei_h, gid_h, u_out, c_out,
            ei_v, gid_v, acc_a, acc_b, node_v, slab, ns_c, nd_c, u_c,
            accG, cntG, slabG, sem_e, sem_g, sem_n,
            mat_a, mat_b, vec_sh, matG, matC):
        s = lax.axis_index("s")
        c = lax.axis_index("c")
        ones16 = jnp.ones((16,), _f32)
        zeros16 = jnp.zeros((16,), _f32)

        def off16(i):
            return pl.ds(pl.multiple_of(i * 16, 16), 16)

        def zero_ref(ref, nv):
            @plsc.parallel_loop(0, nv, 1, unroll=8)
            def _(i):
                ref[off16(i)] = zeros16

        def reduce_rows(mat, nv, out_fn):
            pltpu.sync_copy(mat, slab)

            @plsc.parallel_loop(0, nv, 1, unroll=2)
            def _(i):
                acc = slab[0, off16(i)]
                for r in range(1, NS):
                    acc = acc + slab[r, off16(i)]
                out_fn(i, acc)

        my_nodes = pl.ds(pl.multiple_of(s * CH, 8), CH)

        @pl.when(s < NS - 1)
        def _():
            pltpu.async_copy(
                ei_h.at[:, pl.ds(pl.multiple_of(s * EPT, 512), EPT)], ei_v,
                sem_e)
            pltpu.async_copy(gid_h.at[my_nodes], gid_v, sem_g)

        @pl.when(s == NS - 1)
        def _():
            tail = N - (NS - 1) * CH
            pltpu.async_copy(ei_h.at[:, pl.ds((NS - 1) * EPT, EPT_L)],
                             ei_v.at[:, pl.ds(0, EPT_L)], sem_e)
            pltpu.async_copy(gid_h.at[pl.ds((NS - 1) * CH, tail)],
                             gid_v.at[pl.ds(0, tail)], sem_g)
            for i in range(tail // 16, NV_C):
                gid_v[off16(i)] = jnp.full((16,), G, jnp.int32)

        zero_ref(acc_a, NP // 16)
        zero_ref(acc_b, NP // 16)

        @pl.when(s < NS - 1)
        def _():
            pltpu.make_async_copy(
                ei_h.at[:, pl.ds(pl.multiple_of(s * EPT, 512), EPT)], ei_v,
                sem_e).wait()
            pltpu.make_async_copy(gid_h.at[my_nodes], gid_v, sem_g).wait()

        @pl.when(s == NS - 1)
        def _():
            tail = N - (NS - 1) * CH
            pltpu.make_async_copy(ei_h.at[:, pl.ds((NS - 1) * EPT, EPT_L)],
                                  ei_v.at[:, pl.ds(0, EPT_L)], sem_e).wait()
            pltpu.make_async_copy(gid_h.at[pl.ds((NS - 1) * CH, tail)],
                                  gid_v.at[pl.ds(0, tail)], sem_g).wait()

        def deg_body(i):
            plsc.addupdate_scatter(acc_a, [ei_v[0, off16(i)]], ones16)
            plsc.addupdate_scatter(acc_b, [ei_v[1, off16(i)]], ones16)

        plsc.parallel_loop(0, NV_E_L, 1, unroll=16)(deg_body)

        @pl.when(s < NS - 1)
        def _():
            plsc.parallel_loop(NV_E_L, NV_E, 1, unroll=16)(deg_body)

        pltpu.sync_copy(acc_a, mat_a.at[s])
        pltpu.sync_copy(acc_b, mat_b.at[s])
        plsc.subcore_barrier()

        def ns_fn(i, v):
            ns_c[off16(i)] = _rsqrt16(jnp.maximum(v, 1.0))
        reduce_rows(mat_a.at[:, my_nodes], NV_C, ns_fn)

        def nd_s0_fn(i, v):
            nd = _rsqrt16(jnp.maximum(v, 1.0))
            nd_c[off16(i)] = nd
            u_c[off16(i)] = v * ns_c[off16(i)]
        reduce_rows(mat_b.at[:, my_nodes], NV_C, nd_s0_fn)

        pltpu.sync_copy(u_c, vec_sh.at[my_nodes])
        plsc.subcore_barrier()

        h_n = pltpu.async_copy(vec_sh, node_v, sem_n)
        zero_ref(acc_a, NP // 16)
        h_n.wait()

        def edge_pass():
            def body(i):
                v = plsc.load_gather(node_v, [ei_v[0, off16(i)]])
                plsc.addupdate_scatter(acc_a, [ei_v[1, off16(i)]], v)

            plsc.parallel_loop(0, NV_E_L, 1, unroll=16)(body)

            @pl.when(s < NS - 1)
            def _():
                plsc.parallel_loop(NV_E_L, NV_E, 1, unroll=16)(body)

        edge_pass()
        pltpu.sync_copy(acc_a, mat_a.at[s])
        plsc.subcore_barrier()

        def p_fn(i, v):
            u_c[off16(i)] = v * nd_c[off16(i)] * ns_c[off16(i)]
        reduce_rows(mat_a.at[:, my_nodes], NV_C, p_fn)

        pltpu.sync_copy(u_c, vec_sh.at[my_nodes])
        plsc.subcore_barrier()

        h_n2 = pltpu.async_copy(vec_sh, node_v, sem_n)
        zero_ref(acc_a, NP // 16)
        h_n2.wait()
        edge_pass()

        pltpu.sync_copy(acc_a, mat_a.at[s])
        plsc.subcore_barrier()

        def u_fn(i, v):
            u_c[off16(i)] = v * nd_c[off16(i)]
        reduce_rows(mat_a.at[:, my_nodes], NV_C, u_fn)

        zero_ref(accG, NV_G)
        zero_ref(cntG, NV_G)

        @plsc.parallel_loop(0, NV_C, 1, unroll=4)
        def _(i):
            g = gid_v[off16(i)]
            plsc.addupdate_scatter(accG, [g], u_c[off16(i)])
            plsc.addupdate_scatter(cntG, [g], ones16)

        pltpu.sync_copy(accG, matG.at[s])
        pltpu.sync_copy(cntG, matC.at[s])
        plsc.subcore_barrier()

        @pl.when(jnp.logical_and(s == 0, c == 0))
        def _():
            def redG(mat, out_ref):
                pltpu.sync_copy(mat, slabG)

                @plsc.parallel_loop(0, NV_G, 1, unroll=2)
                def _(i):
                    acc = slabG[0, off16(i)]
                    for r in range(1, NS):
                        acc = acc + slabG[r, off16(i)]
                    out_ref[off16(i)] = acc
            redG(matG, accG)
            redG(matC, cntG)
            pltpu.sync_copy(accG.at[pl.ds(0, G)], u_out)
            pltpu.sync_copy(cntG.at[pl.ds(0, G)], c_out)

    return run


def _tc_bvec(W1, W2, Wfc):
    def body(w1_ref, w2_ref, wfc_ref, o_ref):
        r1 = jnp.maximum(w1_ref[...], 0.0)
        q = jnp.maximum(
            jnp.dot(r1, w2_ref[...], preferred_element_type=_f32), 0.0)
        b = jnp.dot(q, wfc_ref[...], preferred_element_type=_f32)
        o_ref[...] = jnp.concatenate([b, jnp.zeros((1, 16 - C), _f32)], axis=1)

    return pl.pallas_call(
        body, out_shape=jax.ShapeDtypeStruct((1, 16), _f32),
    )(W1, W2, Wfc).reshape(16)


def kernel(edge_index, node_graph_ids, W1, W2, Wfc):
    b = _tc_bvec(W1, W2, Wfc)
    u_sum, cnt = _sc_graph()(edge_index, node_graph_ids.astype(jnp.int32))
    a = u_sum / jnp.maximum(cnt, 1.0)
    return a[:, None] * b[None, :C]


# speedup vs baseline: 65.2060x; 1.0125x over previous
import functools

import jax
import jax.numpy as jnp
from jax import lax
from jax.experimental import pallas as pl
from jax.experimental.pallas import tpu as pltpu
from jax.experimental.pallas import tpu_sc as plsc

N = 10000
E = 160000
H = 256
C = 8
G = 64

NS = 16
NP = 10240
CH = NP // NS
EPT = 10240
EPT_L = E - 15 * EPT
NV_E = EPT // 16
NV_E_L = EPT_L // 16
NV_C = CH // 16
GP = 128
NV_G = GP // 16

_f32 = jnp.float32


def _rsqrt16(x):
    i = plsc.bitcast(x, jnp.int32)
    i = jnp.full((16,), 0x5F3759DF, jnp.int32) - lax.shift_right_logical(
        i, jnp.full((16,), 1, jnp.int32))
    y = plsc.bitcast(i, _f32)
    for _ in range(3):
        y = y * (1.5 - 0.5 * x * y * y)
    return y


def _sc_graph():
    mesh = plsc.VectorSubcoreMesh(
        core_axis_name="c", subcore_axis_name="s", num_cores=2, num_subcores=NS)

    @functools.partial(
        pl.kernel,
        out_type=(jax.ShapeDtypeStruct((G,), _f32),
                  jax.ShapeDtypeStruct((G,), _f32)),
        mesh=mesh,
        compiler_params=pltpu.CompilerParams(needs_layout_passes=False),
        scratch_types=[
            pltpu.VMEM((2, EPT), jnp.int32),
            pltpu.VMEM((CH,), jnp.int32),
            pltpu.VMEM((NP,), _f32),
            pltpu.VMEM((NP,), _f32),
            pltpu.VMEM((NP,), _f32),
            pltpu.VMEM((NS, CH), _f32),
            pltpu.VMEM((NS, CH), _f32),
            pltpu.VMEM((CH,), _f32),
            pltpu.VMEM((CH,), _f32),
            pltpu.VMEM((CH,), _f32),
            pltpu.VMEM((GP,), _f32),
            pltpu.VMEM((GP,), _f32),
            pltpu.VMEM((NS, GP), _f32),
            pltpu.SemaphoreType.DMA,
            pltpu.SemaphoreType.DMA,
            pltpu.SemaphoreType.DMA,
            pltpu.VMEM_SHARED((NS, NP), _f32),
            pltpu.VMEM_SHARED((NS, NP), _f32),
            pltpu.VMEM_SHARED((NP,), _f32),
            pltpu.VMEM_SHARED((NS, GP), _f32),
            pltpu.VMEM_SHARED((NS, GP), _f32),
        ],
    )
    def run(ei_h, gid_h, u_out, c_out,
            ei_v, gid_v, acc_a, acc_b, node_v, slab, slab2, ns_c, nd_c, u_c,
            accG, cntG, slabG, sem_e, sem_g, sem_n,
            mat_a, mat_b, vec_sh, matG, matC):
        s = lax.axis_index("s")
        c = lax.axis_index("c")
        ones16 = jnp.ones((16,), _f32)
        zeros16 = jnp.zeros((16,), _f32)

        def off16(i):
            return pl.ds(pl.multiple_of(i * 16, 16), 16)

        def zero_ref(ref, nv):
            @plsc.parallel_loop(0, nv, 1, unroll=8)
            def _(i):
                ref[off16(i)] = zeros16

        def reduce_rows(mat, nv, out_fn):
            pltpu.sync_copy(mat, slab)

            @plsc.parallel_loop(0, nv, 1, unroll=2)
            def _(i):
                acc = slab[0, off16(i)]
                for r in range(1, NS):
                    acc = acc + slab[r, off16(i)]
                out_fn(i, acc)

        my_nodes = pl.ds(pl.multiple_of(s * CH, 8), CH)

        @pl.when(s < NS - 1)
        def _():
            pltpu.async_copy(
                ei_h.at[:, pl.ds(pl.multiple_of(s * EPT, 512), EPT)], ei_v,
                sem_e)
            pltpu.async_copy(gid_h.at[my_nodes], gid_v, sem_g)

        @pl.when(s == NS - 1)
        def _():
            tail = N - (NS - 1) * CH
            pltpu.async_copy(ei_h.at[:, pl.ds((NS - 1) * EPT, EPT_L)],
                             ei_v.at[:, pl.ds(0, EPT_L)], sem_e)
            pltpu.async_copy(gid_h.at[pl.ds((NS - 1) * CH, tail)],
                             gid_v.at[pl.ds(0, tail)], sem_g)
            for i in range(tail // 16, NV_C):
                gid_v[off16(i)] = jnp.full((16,), G, jnp.int32)

        zero_ref(acc_a, NP // 16)
        zero_ref(acc_b, NP // 16)

        @pl.when(s < NS - 1)
        def _():
            pltpu.make_async_copy(
                ei_h.at[:, pl.ds(pl.multiple_of(s * EPT, 512), EPT)], ei_v,
                sem_e).wait()
            pltpu.make_async_copy(gid_h.at[my_nodes], gid_v, sem_g).wait()

        @pl.when(s == NS - 1)
        def _():
            tail = N - (NS - 1) * CH
            pltpu.make_async_copy(ei_h.at[:, pl.ds((NS - 1) * EPT, EPT_L)],
                                  ei_v.at[:, pl.ds(0, EPT_L)], sem_e).wait()
            pltpu.make_async_copy(gid_h.at[pl.ds((NS - 1) * CH, tail)],
                                  gid_v.at[pl.ds(0, tail)], sem_g).wait()

        def deg_body(i):
            plsc.addupdate_scatter(acc_a, [ei_v[0, off16(i)]], ones16)
            plsc.addupdate_scatter(acc_b, [ei_v[1, off16(i)]], ones16)

        plsc.parallel_loop(0, NV_E_L, 1, unroll=16)(deg_body)

        @pl.when(s < NS - 1)
        def _():
            plsc.parallel_loop(NV_E_L, NV_E, 1, unroll=16)(deg_body)

        pltpu.sync_copy(acc_a, mat_a.at[s])
        pltpu.sync_copy(acc_b, mat_b.at[s])
        plsc.subcore_barrier()

        h_a = pltpu.async_copy(mat_a.at[:, my_nodes], slab, sem_n)
        h_b = pltpu.async_copy(mat_b.at[:, my_nodes], slab2, sem_g)
        h_a.wait()
        h_b.wait()

        @plsc.parallel_loop(0, NV_C, 1, unroll=2)
        def _(i):
            va = slab[0, off16(i)]
            vb = slab2[0, off16(i)]
            for r in range(1, NS):
                va = va + slab[r, off16(i)]
                vb = vb + slab2[r, off16(i)]
            ns = _rsqrt16(jnp.maximum(va, 1.0))
            nd = _rsqrt16(jnp.maximum(vb, 1.0))
            ns_c[off16(i)] = ns
            nd_c[off16(i)] = nd
            u_c[off16(i)] = vb * ns

        pltpu.sync_copy(u_c, vec_sh.at[my_nodes])
        plsc.subcore_barrier()

        h_n = pltpu.async_copy(vec_sh, node_v, sem_n)
        zero_ref(acc_a, NP // 16)
        h_n.wait()

        def edge_pass():
            def body(i):
                v = plsc.load_gather(node_v, [ei_v[0, off16(i)]])
                plsc.addupdate_scatter(acc_a, [ei_v[1, off16(i)]], v)

            plsc.parallel_loop(0, NV_E_L, 1, unroll=16)(body)

            @pl.when(s < NS - 1)
            def _():
                plsc.parallel_loop(NV_E_L, NV_E, 1, unroll=16)(body)

        edge_pass()
        pltpu.sync_copy(acc_a, mat_a.at[s])
        plsc.subcore_barrier()

        def p_fn(i, v):
            u_c[off16(i)] = v * nd_c[off16(i)] * ns_c[off16(i)]
        reduce_rows(mat_a.at[:, my_nodes], NV_C, p_fn)

        pltpu.sync_copy(u_c, vec_sh.at[my_nodes])
        plsc.subcore_barrier()

        h_n2 = pltpu.async_copy(vec_sh, node_v, sem_n)
        zero_ref(acc_a, NP // 16)
        h_n2.wait()
        edge_pass()

        pltpu.sync_copy(acc_a, mat_a.at[s])
        plsc.subcore_barrier()

        def u_fn(i, v):
            u_c[off16(i)] = v * nd_c[off16(i)]
        reduce_rows(mat_a.at[:, my_nodes], NV_C, u_fn)

        zero_ref(accG, NV_G)
        zero_ref(cntG, NV_G)

        @plsc.parallel_loop(0, NV_C, 1, unroll=4)
        def _(i):
            g = gid_v[off16(i)]
            plsc.addupdate_scatter(accG, [g], u_c[off16(i)])
            plsc.addupdate_scatter(cntG, [g], ones16)

        pltpu.sync_copy(accG, matG.at[s])
        pltpu.sync_copy(cntG, matC.at[s])
        plsc.subcore_barrier()

        @pl.when(jnp.logical_and(s == 0, c == 0))
        def _():
            def redG(mat, out_ref):
                pltpu.sync_copy(mat, slabG)

                @plsc.parallel_loop(0, NV_G, 1, unroll=2)
                def _(i):
                    acc = slabG[0, off16(i)]
                    for r in range(1, NS):
                        acc = acc + slabG[r, off16(i)]
                    out_ref[off16(i)] = acc
            redG(matG, accG)
            redG(matC, cntG)
            pltpu.sync_copy(accG.at[pl.ds(0, G)], u_out)
            pltpu.sync_copy(cntG.at[pl.ds(0, G)], c_out)

    return run


def _tc_bvec(W1, W2, Wfc):
    def body(w1_ref, w2_ref, wfc_ref, o_ref):
        r1 = jnp.maximum(w1_ref[...], 0.0)
        q = jnp.maximum(
            jnp.dot(r1, w2_ref[...], preferred_element_type=_f32), 0.0)
        b = jnp.dot(q, wfc_ref[...], preferred_element_type=_f32)
        o_ref[...] = jnp.concatenate([b, jnp.zeros((1, 16 - C), _f32)], axis=1)

    return pl.pallas_call(
        body, out_shape=jax.ShapeDtypeStruct((1, 16), _f32),
    )(W1, W2, Wfc).reshape(16)


def kernel(edge_index, node_graph_ids, W1, W2, Wfc):
    b = _tc_bvec(W1, W2, Wfc)
    u_sum, cnt = _sc_graph()(edge_index, node_graph_ids.astype(jnp.int32))
    a = u_sum / jnp.maximum(cnt, 1.0)
    return a[:, None] * b[None, :C]
